# Initial kernel scaffold; baseline (speedup 1.0000x reference)
import jax
import jax.numpy as jnp
from jax.experimental import pallas as pl


def kernel(x, edge_index, W_in, b_in, Wc0, bc0, g0, be0, Wc1, bc1, g1, be1, Wc2, bc2, g2, be2):
    raise NotImplementedError("write your pallas kernel here")


# speedup vs baseline: 12.6362x; 12.6362x over previous
import functools

import jax
import jax.numpy as jnp
from jax import lax
from jax.experimental import pallas as pl
from jax.experimental.pallas import tpu as pltpu
from jax.experimental.pallas import tpu_sc as plsc

N_NODES = 50000
N_EDGES = 800000
HID = 64
HALF = 32

NUM_CORES = 2
NUM_SUBCORES = 16

CHUNK = 80
CHUNKS_TOTAL = N_EDGES // CHUNK
CHUNKS_PER_TILE = CHUNKS_TOTAL // NUM_SUBCORES
IDX_BLK = 125
N_OUTER = CHUNKS_PER_TILE // IDX_BLK

NPAD = 50176
ROWS_T = NPAD // NUM_SUBCORES
ROWS_W = NPAD // (NUM_CORES * NUM_SUBCORES)

_MESH = plsc.VectorSubcoreMesh(
    core_axis_name="c", subcore_axis_name="s",
    num_cores=NUM_CORES, num_subcores=NUM_SUBCORES)

_SC_PARAMS = pltpu.CompilerParams(use_tc_tiling_on_sc=False)

DEGW = 16


@functools.partial(
    pl.kernel,
    out_type=jax.ShapeDtypeStruct((NPAD, DEGW), jnp.float32),
    mesh=_MESH,
    scratch_types=[
        pltpu.VMEM_SHARED((NPAD, DEGW), jnp.float32),
        pltpu.VMEM((IDX_BLK, CHUNK), jnp.int32),
        pltpu.VMEM((CHUNK, DEGW), jnp.float32),
    ],
    compiler_params=_SC_PARAMS,
)
def _sc_degree(dst2, ones_hbm, zeros_hbm, deg_hbm, accum, didx, ones_v):
    c = lax.axis_index("c")
    s = lax.axis_index("s")
    w = s * NUM_CORES + c

    pltpu.sync_copy(ones_hbm, ones_v)
    tbase = pl.multiple_of(s * ROWS_T, 8)
    pltpu.sync_copy(zeros_hbm, accum.at[pl.ds(tbase, ROWS_T)])
    plsc.subcore_barrier()

    def outer(ob, carry):
        pltpu.sync_copy(dst2.at[s * N_OUTER + ob], didx)

        def inner(j, carry2):
            pltpu.sync_copy(ones_v, accum.at[didx.at[j]], add=True)
            return carry2

        lax.fori_loop(0, IDX_BLK, inner, 0)
        return carry

    lax.fori_loop(0, N_OUTER, outer, 0)
    plsc.subcore_barrier()
    wbase = pl.multiple_of(w * ROWS_W, 8)
    pltpu.sync_copy(accum.at[pl.ds(wbase, ROWS_W)],
                    deg_hbm.at[pl.ds(wbase, ROWS_W)])


@functools.partial(
    pl.kernel,
    out_type=(jax.ShapeDtypeStruct((NPAD, HALF), jnp.float32),
              jax.ShapeDtypeStruct((NPAD, HALF), jnp.float32)),
    mesh=_MESH,
    scratch_types=[
        pltpu.VMEM_SHARED((NPAD, HALF), jnp.float32),
        pltpu.VMEM((IDX_BLK, CHUNK), jnp.int32),
        pltpu.VMEM((IDX_BLK, CHUNK), jnp.int32),
        pltpu.VMEM((CHUNK, HALF), jnp.float32),
        pltpu.SemaphoreType.DMA,
    ],
    compiler_params=_SC_PARAMS,
)
def _sc_segment(y_lo, y_hi, src2, dst2, agg_lo, agg_hi,
                accum, sidx, didx, rows, gsem):
    c = lax.axis_index("c")
    s = lax.axis_index("s")
    tbase = pl.multiple_of(s * ROWS_T, 8)

    def run(y_ref, agg_ref):
        pltpu.sync_copy(y_ref.at[pl.ds(tbase, ROWS_T)],
                        accum.at[pl.ds(tbase, ROWS_T)])
        plsc.subcore_barrier()

        def outer(ob, carry):
            pltpu.sync_copy(src2.at[s * N_OUTER + ob], sidx)
            pltpu.sync_copy(dst2.at[s * N_OUTER + ob], didx)

            def inner(j, carry2):
                pltpu.async_copy(y_ref.at[sidx.at[j]], rows, gsem).wait()
                pltpu.sync_copy(rows, accum.at[didx.at[j]], add=True)
                return carry2

            lax.fori_loop(0, IDX_BLK, inner, 0)
            return carry

        lax.fori_loop(0, N_OUTER, outer, 0)
        plsc.subcore_barrier()
        pltpu.sync_copy(accum.at[pl.ds(tbase, ROWS_T)],
                        agg_ref.at[pl.ds(tbase, ROWS_T)])

    @pl.when(c == 0)
    def _():
        run(y_lo, agg_lo)

    @pl.when(c == 1)
    def _():
        run(y_hi, agg_hi)


BLK = 2000
GRID = N_NODES // BLK

_row_spec = lambda w: pl.BlockSpec((BLK, w), lambda i: (i, 0))
_full_spec = lambda r, w: pl.BlockSpec((r, w), lambda i: (0, 0))


def _tc_proj_body(x_ref, win_ref, bin_ref, wc0_ref, deg_ref,
                  h_ref, dinv_ref, ylo_ref, yhi_ref):
    h = jnp.dot(x_ref[...], win_ref[...],
                preferred_element_type=jnp.float32) + bin_ref[...]
    dinv = lax.rsqrt(deg_ref[...][:, 0:1] + 1.0)
    y = jnp.dot(h, wc0_ref[...], preferred_element_type=jnp.float32) * dinv
    h_ref[...] = h
    dinv_ref[...] = dinv
    ylo_ref[...] = y[:, :HALF]
    yhi_ref[...] = y[:, HALF:]


_tc_proj = pl.pallas_call(
    _tc_proj_body,
    grid=(GRID,),
    in_specs=[_row_spec(5), _full_spec(5, HID), _full_spec(1, HID),
              _full_spec(HID, HID), _row_spec(DEGW)],
    out_specs=(_row_spec(HID), _row_spec(1), _row_spec(HALF), _row_spec(HALF)),
    out_shape=(jax.ShapeDtypeStruct((N_NODES, HID), jnp.float32),
               jax.ShapeDtypeStruct((N_NODES, 1), jnp.float32),
               jax.ShapeDtypeStruct((N_NODES, HALF), jnp.float32),
               jax.ShapeDtypeStruct((N_NODES, HALF), jnp.float32)),
)


def _layer_epilogue(h, agg, dinv, bc, g, be, relu):
    t = dinv * agg + bc
    mu = jnp.mean(t, axis=-1, keepdims=True)
    var = jnp.mean((t - mu) ** 2, axis=-1, keepdims=True)
    t = (t - mu) * lax.rsqrt(var + 1e-5) * g + be
    if relu:
        t = jnp.maximum(t, 0.0)
    return h + t


def _tc_mid_body(h_ref, alo_ref, ahi_ref, dinv_ref, bc_ref, g_ref, be_ref,
                 wn_ref, hn_ref, ylo_ref, yhi_ref):
    agg = jnp.concatenate([alo_ref[...], ahi_ref[...]], axis=1)
    dinv = dinv_ref[...]
    h_new = _layer_epilogue(h_ref[...], agg, dinv, bc_ref[...], g_ref[...],
                            be_ref[...], relu=True)
    y = jnp.dot(h_new, wn_ref[...], preferred_element_type=jnp.float32) * dinv
    hn_ref[...] = h_new
    ylo_ref[...] = y[:, :HALF]
    yhi_ref[...] = y[:, HALF:]


_tc_mid = pl.pallas_call(
    _tc_mid_body,
    grid=(GRID,),
    in_specs=[_row_spec(HID), _row_spec(HALF), _row_spec(HALF), _row_spec(1),
              _full_spec(1, HID), _full_spec(1, HID), _full_spec(1, HID),
              _full_spec(HID, HID)],
    out_specs=(_row_spec(HID), _row_spec(HALF), _row_spec(HALF)),
    out_shape=(jax.ShapeDtypeStruct((N_NODES, HID), jnp.float32),
               jax.ShapeDtypeStruct((N_NODES, HALF), jnp.float32),
               jax.ShapeDtypeStruct((N_NODES, HALF), jnp.float32)),
)


def _tc_fin_body(h_ref, alo_ref, ahi_ref, dinv_ref, bc_ref, g_ref, be_ref,
                 hn_ref):
    agg = jnp.concatenate([alo_ref[...], ahi_ref[...]], axis=1)
    hn_ref[...] = _layer_epilogue(h_ref[...], agg, dinv_ref[...], bc_ref[...],
                                  g_ref[...], be_ref[...], relu=False)


_tc_fin = pl.pallas_call(
    _tc_fin_body,
    grid=(GRID,),
    in_specs=[_row_spec(HID), _row_spec(HALF), _row_spec(HALF), _row_spec(1),
              _full_spec(1, HID), _full_spec(1, HID), _full_spec(1, HID)],
    out_specs=_row_spec(HID),
    out_shape=jax.ShapeDtypeStruct((N_NODES, HID), jnp.float32),
)


def kernel(x, edge_index, W_in, b_in, Wc0, bc0, g0, be0,
           Wc1, bc1, g1, be1, Wc2, bc2, g2, be2):
    src2 = edge_index[0].reshape(NUM_SUBCORES * N_OUTER, IDX_BLK, CHUNK)
    dst2 = edge_index[1].reshape(NUM_SUBCORES * N_OUTER, IDX_BLK, CHUNK)
    pad = ((0, NPAD - N_NODES), (0, 0))

    ones16 = jnp.ones((CHUNK, DEGW), jnp.float32)
    zeros16 = jnp.zeros((ROWS_T, DEGW), jnp.float32)
    deg = _sc_degree(dst2, ones16, zeros16)
    h, dinv, ylo, yhi = _tc_proj(x, W_in, b_in.reshape(1, HID), Wc0, deg)

    params = ((bc0, g0, be0, Wc1), (bc1, g1, be1, Wc2), (bc2, g2, be2, None))
    for i, (bc, g, be, wnext) in enumerate(params):
        agg_lo, agg_hi = _sc_segment(jnp.pad(ylo, pad), jnp.pad(yhi, pad),
                                     src2, dst2)
        bc2d, g2d, be2d = (bc.reshape(1, HID), g.reshape(1, HID),
                           be.reshape(1, HID))
        if wnext is not None:
            h, ylo, yhi = _tc_mid(h, agg_lo, agg_hi, dinv, bc2d, g2d, be2d,
                                  wnext)
        else:
            h = _tc_fin(h, agg_lo, agg_hi, dinv, bc2d, g2d, be2d)
    return h


# speedup vs baseline: 19.8917x; 1.5742x over previous
import functools

import jax
import jax.numpy as jnp
from jax import lax
from jax.experimental import pallas as pl
from jax.experimental.pallas import tpu as pltpu
from jax.experimental.pallas import tpu_sc as plsc

N_NODES = 50000
N_EDGES = 800000
HID = 64
HALF = 32

NUM_CORES = 2
NUM_SUBCORES = 16

CHUNK = 80
CHUNKS_TOTAL = N_EDGES // CHUNK
CHUNKS_PER_TILE = CHUNKS_TOTAL // NUM_SUBCORES
IDX_BLK = 125
N_OUTER = CHUNKS_PER_TILE // IDX_BLK

NPAD = 50176
ROWS_T = NPAD // NUM_SUBCORES
ROWS_W = NPAD // (NUM_CORES * NUM_SUBCORES)

_MESH = plsc.VectorSubcoreMesh(
    core_axis_name="c", subcore_axis_name="s",
    num_cores=NUM_CORES, num_subcores=NUM_SUBCORES)

_SC_PARAMS = pltpu.CompilerParams(use_tc_tiling_on_sc=False)

DEGW = 16


@functools.partial(
    pl.kernel,
    out_type=jax.ShapeDtypeStruct((NPAD, DEGW), jnp.float32),
    mesh=_MESH,
    scratch_types=[
        pltpu.VMEM_SHARED((NPAD, DEGW), jnp.float32),
        pltpu.VMEM((IDX_BLK, CHUNK), jnp.int32),
        pltpu.VMEM((CHUNK, DEGW), jnp.float32),
    ],
    compiler_params=_SC_PARAMS,
)
def _sc_degree(dst2, ones_hbm, zeros_hbm, deg_hbm, accum, didx, ones_v):
    c = lax.axis_index("c")
    s = lax.axis_index("s")
    w = s * NUM_CORES + c

    pltpu.sync_copy(ones_hbm, ones_v)
    tbase = pl.multiple_of(s * ROWS_T, 8)
    pltpu.sync_copy(zeros_hbm, accum.at[pl.ds(tbase, ROWS_T)])
    plsc.subcore_barrier()

    def outer(ob, carry):
        pltpu.sync_copy(dst2.at[s * N_OUTER + ob], didx)

        def inner(j, carry2):
            pltpu.sync_copy(ones_v, accum.at[didx.at[j]], add=True)
            return carry2

        lax.fori_loop(0, IDX_BLK, inner, 0)
        return carry

    lax.fori_loop(0, N_OUTER, outer, 0)
    plsc.subcore_barrier()
    wbase = pl.multiple_of(w * ROWS_W, 8)
    pltpu.sync_copy(accum.at[pl.ds(wbase, ROWS_W)],
                    deg_hbm.at[pl.ds(wbase, ROWS_W)])


@functools.partial(
    pl.kernel,
    out_type=(jax.ShapeDtypeStruct((NPAD, HALF), jnp.float32),
              jax.ShapeDtypeStruct((NPAD, HALF), jnp.float32)),
    mesh=_MESH,
    scratch_types=[
        pltpu.VMEM_SHARED((NPAD, HALF), jnp.float32),
        pltpu.VMEM((IDX_BLK, CHUNK), jnp.int32),
        pltpu.VMEM((IDX_BLK, CHUNK), jnp.int32),
        pltpu.VMEM((CHUNK, HALF), jnp.float32),
        pltpu.VMEM((CHUNK, HALF), jnp.float32),
        pltpu.SemaphoreType.DMA,
        pltpu.SemaphoreType.DMA,
    ],
    compiler_params=_SC_PARAMS,
)
def _sc_segment(y_lo, y_hi, src2, dst2, agg_lo, agg_hi,
                accum, sidx, didx, rows_a, rows_b, sem_a, sem_b):
    c = lax.axis_index("c")
    s = lax.axis_index("s")
    tbase = pl.multiple_of(s * ROWS_T, 8)

    def run(y_ref, agg_ref):
        pltpu.sync_copy(y_ref.at[pl.ds(tbase, ROWS_T)],
                        accum.at[pl.ds(tbase, ROWS_T)])
        plsc.subcore_barrier()

        def outer(ob, carry):
            pltpu.sync_copy(src2.at[s * N_OUTER + ob], sidx)
            pltpu.sync_copy(dst2.at[s * N_OUTER + ob], didx)
            pltpu.async_copy(y_ref.at[sidx.at[0]], rows_a, sem_a)

            def inner(jj, carry2):
                j = jj * 2
                pltpu.async_copy(y_ref.at[sidx.at[j + 1]], rows_b, sem_b)
                pltpu.make_async_copy(y_ref.at[sidx.at[j]], rows_a,
                                      sem_a).wait()
                pltpu.sync_copy(rows_a, accum.at[didx.at[j]], add=True)
                pltpu.async_copy(y_ref.at[sidx.at[j + 2]], rows_a, sem_a)
                pltpu.make_async_copy(y_ref.at[sidx.at[j + 1]], rows_b,
                                      sem_b).wait()
                pltpu.sync_copy(rows_b, accum.at[didx.at[j + 1]], add=True)
                return carry2

            lax.fori_loop(0, (IDX_BLK - 1) // 2, inner, 0)
            pltpu.make_async_copy(y_ref.at[sidx.at[IDX_BLK - 1]], rows_a,
                                  sem_a).wait()
            pltpu.sync_copy(rows_a, accum.at[didx.at[IDX_BLK - 1]], add=True)
            return carry

        lax.fori_loop(0, N_OUTER, outer, 0)
        plsc.subcore_barrier()
        pltpu.sync_copy(accum.at[pl.ds(tbase, ROWS_T)],
                        agg_ref.at[pl.ds(tbase, ROWS_T)])

    @pl.when(c == 0)
    def _():
        run(y_lo, agg_lo)

    @pl.when(c == 1)
    def _():
        run(y_hi, agg_hi)


BLK = 2000
GRID = N_NODES // BLK

_row_spec = lambda w: pl.BlockSpec((BLK, w), lambda i: (i, 0))
_full_spec = lambda r, w: pl.BlockSpec((r, w), lambda i: (0, 0))


def _tc_proj_body(x_ref, win_ref, bin_ref, wc0_ref, deg_ref,
                  h_ref, dinv_ref, ylo_ref, yhi_ref):
    h = jnp.dot(x_ref[...], win_ref[...],
                preferred_element_type=jnp.float32) + bin_ref[...]
    dinv = lax.rsqrt(deg_ref[...][:, 0:1] + 1.0)
    y = jnp.dot(h, wc0_ref[...], preferred_element_type=jnp.float32) * dinv
    h_ref[...] = h
    dinv_ref[...] = dinv
    ylo_ref[...] = y[:, :HALF]
    yhi_ref[...] = y[:, HALF:]


_tc_proj = pl.pallas_call(
    _tc_proj_body,
    grid=(GRID,),
    in_specs=[_row_spec(5), _full_spec(5, HID), _full_spec(1, HID),
              _full_spec(HID, HID), _row_spec(DEGW)],
    out_specs=(_row_spec(HID), _row_spec(1), _row_spec(HALF), _row_spec(HALF)),
    out_shape=(jax.ShapeDtypeStruct((N_NODES, HID), jnp.float32),
               jax.ShapeDtypeStruct((N_NODES, 1), jnp.float32),
               jax.ShapeDtypeStruct((NPAD, HALF), jnp.float32),
               jax.ShapeDtypeStruct((NPAD, HALF), jnp.float32)),
)


def _layer_epilogue(h, agg, dinv, bc, g, be, relu):
    t = dinv * agg + bc
    mu = jnp.mean(t, axis=-1, keepdims=True)
    var = jnp.mean((t - mu) ** 2, axis=-1, keepdims=True)
    t = (t - mu) * lax.rsqrt(var + 1e-5) * g + be
    if relu:
        t = jnp.maximum(t, 0.0)
    return h + t


def _tc_mid_body(h_ref, alo_ref, ahi_ref, dinv_ref, bc_ref, g_ref, be_ref,
                 wn_ref, hn_ref, ylo_ref, yhi_ref):
    agg = jnp.concatenate([alo_ref[...], ahi_ref[...]], axis=1)
    dinv = dinv_ref[...]
    h_new = _layer_epilogue(h_ref[...], agg, dinv, bc_ref[...], g_ref[...],
                            be_ref[...], relu=True)
    y = jnp.dot(h_new, wn_ref[...], preferred_element_type=jnp.float32) * dinv
    hn_ref[...] = h_new
    ylo_ref[...] = y[:, :HALF]
    yhi_ref[...] = y[:, HALF:]


_tc_mid = pl.pallas_call(
    _tc_mid_body,
    grid=(GRID,),
    in_specs=[_row_spec(HID), _row_spec(HALF), _row_spec(HALF), _row_spec(1),
              _full_spec(1, HID), _full_spec(1, HID), _full_spec(1, HID),
              _full_spec(HID, HID)],
    out_specs=(_row_spec(HID), _row_spec(HALF), _row_spec(HALF)),
    out_shape=(jax.ShapeDtypeStruct((N_NODES, HID), jnp.float32),
               jax.ShapeDtypeStruct((NPAD, HALF), jnp.float32),
               jax.ShapeDtypeStruct((NPAD, HALF), jnp.float32)),
)


def _tc_fin_body(h_ref, alo_ref, ahi_ref, dinv_ref, bc_ref, g_ref, be_ref,
                 hn_ref):
    agg = jnp.concatenate([alo_ref[...], ahi_ref[...]], axis=1)
    hn_ref[...] = _layer_epilogue(h_ref[...], agg, dinv_ref[...], bc_ref[...],
                                  g_ref[...], be_ref[...], relu=False)


_tc_fin = pl.pallas_call(
    _tc_fin_body,
    grid=(GRID,),
    in_specs=[_row_spec(HID), _row_spec(HALF), _row_spec(HALF), _row_spec(1),
              _full_spec(1, HID), _full_spec(1, HID), _full_spec(1, HID)],
    out_specs=_row_spec(HID),
    out_shape=jax.ShapeDtypeStruct((N_NODES, HID), jnp.float32),
)


def kernel(x, edge_index, W_in, b_in, Wc0, bc0, g0, be0,
           Wc1, bc1, g1, be1, Wc2, bc2, g2, be2):
    src2 = edge_index[0].reshape(NUM_SUBCORES * N_OUTER, IDX_BLK, CHUNK)
    dst2 = edge_index[1].reshape(NUM_SUBCORES * N_OUTER, IDX_BLK, CHUNK)

    ones16 = jnp.ones((CHUNK, DEGW), jnp.float32)
    zeros16 = jnp.zeros((ROWS_T, DEGW), jnp.float32)
    deg = _sc_degree(dst2, ones16, zeros16)
    h, dinv, ylo, yhi = _tc_proj(x, W_in, b_in.reshape(1, HID), Wc0, deg)

    params = ((bc0, g0, be0, Wc1), (bc1, g1, be1, Wc2), (bc2, g2, be2, None))
    for i, (bc, g, be, wnext) in enumerate(params):
        agg_lo, agg_hi = _sc_segment(ylo, yhi, src2, dst2)
        bc2d, g2d, be2d = (bc.reshape(1, HID), g.reshape(1, HID),
                           be.reshape(1, HID))
        if wnext is not None:
            h, ylo, yhi = _tc_mid(h, agg_lo, agg_hi, dinv, bc2d, g2d, be2d,
                                  wnext)
        else:
            h = _tc_fin(h, agg_lo, agg_hi, dinv, bc2d, g2d, be2d)
    return h


# speedup vs baseline: 26.4173x; 1.3281x over previous
import functools

import jax
import jax.numpy as jnp
from jax import lax
from jax.experimental import pallas as pl
from jax.experimental.pallas import tpu as pltpu
from jax.experimental.pallas import tpu_sc as plsc

N_NODES = 50000
N_EDGES = 800000
HID = 64
HALF = 32

NUM_CORES = 2
NUM_SUBCORES = 16

CHUNK = 80
CHUNKS_TOTAL = N_EDGES // CHUNK
CHUNKS_PER_TILE = CHUNKS_TOTAL // NUM_SUBCORES
IDX_BLK = 125
N_OUTER = CHUNKS_PER_TILE // IDX_BLK

NPAD = 50176
ROWS_T = NPAD // NUM_SUBCORES
ROWS_W = NPAD // (NUM_CORES * NUM_SUBCORES)

_MESH = plsc.VectorSubcoreMesh(
    core_axis_name="c", subcore_axis_name="s",
    num_cores=NUM_CORES, num_subcores=NUM_SUBCORES)

_SC_PARAMS = pltpu.CompilerParams(use_tc_tiling_on_sc=False)

DEGW = 16


@functools.partial(
    pl.kernel,
    out_type=jax.ShapeDtypeStruct((NPAD, DEGW), jnp.float32),
    mesh=_MESH,
    scratch_types=[
        pltpu.VMEM_SHARED((NPAD, DEGW), jnp.float32),
        pltpu.VMEM((IDX_BLK, CHUNK), jnp.int32),
        pltpu.VMEM((CHUNK, DEGW), jnp.float32),
    ],
    compiler_params=_SC_PARAMS,
)
def _sc_degree(dst2, ones_hbm, zeros_hbm, deg_hbm, accum, didx, ones_v):
    c = lax.axis_index("c")
    s = lax.axis_index("s")
    w = s * NUM_CORES + c

    pltpu.sync_copy(ones_hbm, ones_v)
    tbase = pl.multiple_of(s * ROWS_T, 8)
    pltpu.sync_copy(zeros_hbm, accum.at[pl.ds(tbase, ROWS_T)])
    plsc.subcore_barrier()

    def outer(ob, carry):
        pltpu.sync_copy(dst2.at[s * N_OUTER + ob], didx)

        def inner(j, carry2):
            pltpu.sync_copy(ones_v, accum.at[didx.at[j]], add=True)
            return carry2

        lax.fori_loop(0, IDX_BLK, inner, 0)
        return carry

    lax.fori_loop(0, N_OUTER, outer, 0)
    plsc.subcore_barrier()
    wbase = pl.multiple_of(w * ROWS_W, 8)
    pltpu.sync_copy(accum.at[pl.ds(wbase, ROWS_W)],
                    deg_hbm.at[pl.ds(wbase, ROWS_W)])


@functools.partial(
    pl.kernel,
    out_type=(jax.ShapeDtypeStruct((NPAD, HALF), jnp.float32),
              jax.ShapeDtypeStruct((NPAD, HALF), jnp.float32)),
    mesh=_MESH,
    scratch_types=[
        pltpu.VMEM_SHARED((NPAD, HALF), jnp.float32),
        pltpu.VMEM((IDX_BLK, CHUNK), jnp.int32),
        pltpu.VMEM((IDX_BLK, CHUNK), jnp.int32),
        pltpu.VMEM((CHUNK, HALF), jnp.float32),
        pltpu.VMEM((CHUNK, HALF), jnp.float32),
        pltpu.VMEM((CHUNK, HALF), jnp.float32),
        pltpu.VMEM((CHUNK, HALF), jnp.float32),
        pltpu.SemaphoreType.DMA,
        pltpu.SemaphoreType.DMA,
        pltpu.SemaphoreType.DMA,
        pltpu.SemaphoreType.DMA,
    ],
    compiler_params=_SC_PARAMS,
)
def _sc_segment(y_lo, y_hi, src2, dst2, agg_lo, agg_hi, accum, sidx, didx,
                rows_a, rows_b, rows_c, rows_d, sem_a, sem_b, sem_c, sem_d):
    c = lax.axis_index("c")
    s = lax.axis_index("s")
    tbase = pl.multiple_of(s * ROWS_T, 8)
    bufs = ((rows_a, sem_a), (rows_b, sem_b), (rows_c, sem_c), (rows_d, sem_d))

    def run(y_ref, agg_ref):
        pltpu.sync_copy(y_ref.at[pl.ds(tbase, ROWS_T)],
                        accum.at[pl.ds(tbase, ROWS_T)])
        plsc.subcore_barrier()

        def gather(k, slot):
            buf, sem = bufs[slot]
            pltpu.async_copy(y_ref.at[sidx.at[k]], buf, sem)

        def drain(k, slot):
            buf, sem = bufs[slot]
            pltpu.make_async_copy(y_ref.at[sidx.at[k]], buf, sem).wait()
            pltpu.sync_copy(buf, accum.at[didx.at[k]], add=True)

        def outer(ob, carry):
            pltpu.sync_copy(src2.at[s * N_OUTER + ob], sidx)
            pltpu.sync_copy(dst2.at[s * N_OUTER + ob], didx)
            gather(0, 0)
            gather(1, 1)
            gather(2, 2)

            def inner(jj, carry2):
                j = jj * 4
                gather(j + 3, 3)
                drain(j, 0)
                gather(j + 4, 0)
                drain(j + 1, 1)
                gather(j + 5, 1)
                drain(j + 2, 2)
                gather(j + 6, 2)
                drain(j + 3, 3)
                return carry2

            lax.fori_loop(0, (IDX_BLK - 5) // 4, inner, 0)
            gather(IDX_BLK - 2, 3)
            drain(IDX_BLK - 5, 0)
            gather(IDX_BLK - 1, 0)
            drain(IDX_BLK - 4, 1)
            drain(IDX_BLK - 3, 2)
            drain(IDX_BLK - 2, 3)
            drain(IDX_BLK - 1, 0)
            return carry

        lax.fori_loop(0, N_OUTER, outer, 0)
        plsc.subcore_barrier()
        pltpu.sync_copy(accum.at[pl.ds(tbase, ROWS_T)],
                        agg_ref.at[pl.ds(tbase, ROWS_T)])

    @pl.when(c == 0)
    def _():
        run(y_lo, agg_lo)

    @pl.when(c == 1)
    def _():
        run(y_hi, agg_hi)


BLK = 2000
GRID = N_NODES // BLK

_row_spec = lambda w: pl.BlockSpec((BLK, w), lambda i: (i, 0))
_full_spec = lambda r, w: pl.BlockSpec((r, w), lambda i: (0, 0))


def _tc_proj_body(x_ref, win_ref, bin_ref, wc0_ref, deg_ref,
                  h_ref, dinv_ref, ylo_ref, yhi_ref):
    h = jnp.dot(x_ref[...], win_ref[...],
                preferred_element_type=jnp.float32) + bin_ref[...]
    dinv = lax.rsqrt(deg_ref[...][:, 0:1] + 1.0)
    y = jnp.dot(h, wc0_ref[...], preferred_element_type=jnp.float32) * dinv
    h_ref[...] = h
    dinv_ref[...] = dinv
    ylo_ref[...] = y[:, :HALF]
    yhi_ref[...] = y[:, HALF:]


_tc_proj = pl.pallas_call(
    _tc_proj_body,
    grid=(GRID,),
    in_specs=[_row_spec(5), _full_spec(5, HID), _full_spec(1, HID),
              _full_spec(HID, HID), _row_spec(DEGW)],
    out_specs=(_row_spec(HID), _row_spec(1), _row_spec(HALF), _row_spec(HALF)),
    out_shape=(jax.ShapeDtypeStruct((N_NODES, HID), jnp.float32),
               jax.ShapeDtypeStruct((N_NODES, 1), jnp.float32),
               jax.ShapeDtypeStruct((NPAD, HALF), jnp.float32),
               jax.ShapeDtypeStruct((NPAD, HALF), jnp.float32)),
)


def _layer_epilogue(h, agg, dinv, bc, g, be, relu):
    t = dinv * agg + bc
    mu = jnp.mean(t, axis=-1, keepdims=True)
    var = jnp.mean((t - mu) ** 2, axis=-1, keepdims=True)
    t = (t - mu) * lax.rsqrt(var + 1e-5) * g + be
    if relu:
        t = jnp.maximum(t, 0.0)
    return h + t


def _tc_mid_body(h_ref, alo_ref, ahi_ref, dinv_ref, bc_ref, g_ref, be_ref,
                 wn_ref, hn_ref, ylo_ref, yhi_ref):
    agg = jnp.concatenate([alo_ref[...], ahi_ref[...]], axis=1)
    dinv = dinv_ref[...]
    h_new = _layer_epilogue(h_ref[...], agg, dinv, bc_ref[...], g_ref[...],
                            be_ref[...], relu=True)
    y = jnp.dot(h_new, wn_ref[...], preferred_element_type=jnp.float32) * dinv
    hn_ref[...] = h_new
    ylo_ref[...] = y[:, :HALF]
    yhi_ref[...] = y[:, HALF:]


_tc_mid = pl.pallas_call(
    _tc_mid_body,
    grid=(GRID,),
    in_specs=[_row_spec(HID), _row_spec(HALF), _row_spec(HALF), _row_spec(1),
              _full_spec(1, HID), _full_spec(1, HID), _full_spec(1, HID),
              _full_spec(HID, HID)],
    out_specs=(_row_spec(HID), _row_spec(HALF), _row_spec(HALF)),
    out_shape=(jax.ShapeDtypeStruct((N_NODES, HID), jnp.float32),
               jax.ShapeDtypeStruct((NPAD, HALF), jnp.float32),
               jax.ShapeDtypeStruct((NPAD, HALF), jnp.float32)),
)


def _tc_fin_body(h_ref, alo_ref, ahi_ref, dinv_ref, bc_ref, g_ref, be_ref,
                 hn_ref):
    agg = jnp.concatenate([alo_ref[...], ahi_ref[...]], axis=1)
    hn_ref[...] = _layer_epilogue(h_ref[...], agg, dinv_ref[...], bc_ref[...],
                                  g_ref[...], be_ref[...], relu=False)


_tc_fin = pl.pallas_call(
    _tc_fin_body,
    grid=(GRID,),
    in_specs=[_row_spec(HID), _row_spec(HALF), _row_spec(HALF), _row_spec(1),
              _full_spec(1, HID), _full_spec(1, HID), _full_spec(1, HID)],
    out_specs=_row_spec(HID),
    out_shape=jax.ShapeDtypeStruct((N_NODES, HID), jnp.float32),
)


def kernel(x, edge_index, W_in, b_in, Wc0, bc0, g0, be0,
           Wc1, bc1, g1, be1, Wc2, bc2, g2, be2):
    src2 = edge_index[0].reshape(NUM_SUBCORES * N_OUTER, IDX_BLK, CHUNK)
    dst2 = edge_index[1].reshape(NUM_SUBCORES * N_OUTER, IDX_BLK, CHUNK)

    ones16 = jnp.ones((CHUNK, DEGW), jnp.float32)
    zeros16 = jnp.zeros((ROWS_T, DEGW), jnp.float32)
    deg = _sc_degree(dst2, ones16, zeros16)
    h, dinv, ylo, yhi = _tc_proj(x, W_in, b_in.reshape(1, HID), Wc0, deg)

    params = ((bc0, g0, be0, Wc1), (bc1, g1, be1, Wc2), (bc2, g2, be2, None))
    for i, (bc, g, be, wnext) in enumerate(params):
        agg_lo, agg_hi = _sc_segment(ylo, yhi, src2, dst2)
        bc2d, g2d, be2d = (bc.reshape(1, HID), g.reshape(1, HID),
                           be.reshape(1, HID))
        if wnext is not None:
            h, ylo, yhi = _tc_mid(h, agg_lo, agg_hi, dinv, bc2d, g2d, be2d,
                                  wnext)
        else:
            h = _tc_fin(h, agg_lo, agg_hi, dinv, bc2d, g2d, be2d)
    return h


# speedup vs baseline: 27.2049x; 1.0298x over previous
import functools

import jax
import jax.numpy as jnp
from jax import lax
from jax.experimental import pallas as pl
from jax.experimental.pallas import tpu as pltpu
from jax.experimental.pallas import tpu_sc as plsc

N_NODES = 50000
N_EDGES = 800000
HID = 64
HALF = 32

NUM_CORES = 2
NUM_SUBCORES = 16

CHUNK = 128
E_PAD = 802816
CHUNKS_TOTAL = E_PAD // CHUNK
CHUNKS_PER_TILE = CHUNKS_TOTAL // NUM_SUBCORES
IDX_BLK = 49
N_OUTER = CHUNKS_PER_TILE // IDX_BLK

NPAD = 50176
ROWS_T = NPAD // NUM_SUBCORES
ROWS_W = NPAD // (NUM_CORES * NUM_SUBCORES)

_MESH = plsc.VectorSubcoreMesh(
    core_axis_name="c", subcore_axis_name="s",
    num_cores=NUM_CORES, num_subcores=NUM_SUBCORES)

_SC_PARAMS = pltpu.CompilerParams(use_tc_tiling_on_sc=False)

DEGW = 16


@functools.partial(
    pl.kernel,
    out_type=jax.ShapeDtypeStruct((NPAD, DEGW), jnp.float32),
    mesh=_MESH,
    scratch_types=[
        pltpu.VMEM_SHARED((NPAD, DEGW), jnp.float32),
        pltpu.VMEM((IDX_BLK, CHUNK), jnp.int32),
        pltpu.VMEM((CHUNK, DEGW), jnp.float32),
    ],
    compiler_params=_SC_PARAMS,
)
def _sc_degree(dst2, ones_hbm, zeros_hbm, deg_hbm, accum, didx, ones_v):
    c = lax.axis_index("c")
    s = lax.axis_index("s")
    w = s * NUM_CORES + c

    pltpu.sync_copy(ones_hbm, ones_v)
    tbase = pl.multiple_of(s * ROWS_T, 8)
    pltpu.sync_copy(zeros_hbm, accum.at[pl.ds(tbase, ROWS_T)])
    plsc.subcore_barrier()

    def outer(ob, carry):
        pltpu.sync_copy(dst2.at[s * N_OUTER + ob], didx)

        def inner(j, carry2):
            pltpu.sync_copy(ones_v, accum.at[didx.at[j]], add=True)
            return carry2

        lax.fori_loop(0, IDX_BLK, inner, 0)
        return carry

    lax.fori_loop(0, N_OUTER, outer, 0)
    plsc.subcore_barrier()
    wbase = pl.multiple_of(w * ROWS_W, 8)
    pltpu.sync_copy(accum.at[pl.ds(wbase, ROWS_W)],
                    deg_hbm.at[pl.ds(wbase, ROWS_W)])


@functools.partial(
    pl.kernel,
    out_type=(jax.ShapeDtypeStruct((NPAD, HALF), jnp.float32),
              jax.ShapeDtypeStruct((NPAD, HALF), jnp.float32)),
    mesh=_MESH,
    scratch_types=[
        pltpu.VMEM_SHARED((NPAD, HALF), jnp.float32),
        pltpu.VMEM((IDX_BLK, CHUNK), jnp.int32),
        pltpu.VMEM((IDX_BLK, CHUNK), jnp.int32),
        pltpu.VMEM((CHUNK, HALF), jnp.float32),
        pltpu.VMEM((CHUNK, HALF), jnp.float32),
        pltpu.VMEM((CHUNK, HALF), jnp.float32),
        pltpu.VMEM((CHUNK, HALF), jnp.float32),
        pltpu.SemaphoreType.DMA,
        pltpu.SemaphoreType.DMA,
        pltpu.SemaphoreType.DMA,
        pltpu.SemaphoreType.DMA,
    ],
    compiler_params=_SC_PARAMS,
)
def _sc_segment(y_lo, y_hi, src2, dst2, agg_lo, agg_hi, accum, sidx, didx,
                rows_a, rows_b, rows_c, rows_d, sem_a, sem_b, sem_c, sem_d):
    c = lax.axis_index("c")
    s = lax.axis_index("s")
    tbase = pl.multiple_of(s * ROWS_T, 8)
    bufs = ((rows_a, sem_a), (rows_b, sem_b), (rows_c, sem_c), (rows_d, sem_d))

    def run(y_ref, agg_ref):
        pltpu.sync_copy(y_ref.at[pl.ds(tbase, ROWS_T)],
                        accum.at[pl.ds(tbase, ROWS_T)])
        plsc.subcore_barrier()

        def gather(k, slot):
            buf, sem = bufs[slot]
            pltpu.async_copy(y_ref.at[sidx.at[k]], buf, sem)

        def drain(k, slot):
            buf, sem = bufs[slot]
            pltpu.make_async_copy(y_ref.at[sidx.at[k]], buf, sem).wait()
            pltpu.sync_copy(buf, accum.at[didx.at[k]], add=True)

        def outer(ob, carry):
            pltpu.sync_copy(src2.at[s * N_OUTER + ob], sidx)
            pltpu.sync_copy(dst2.at[s * N_OUTER + ob], didx)
            gather(0, 0)
            gather(1, 1)
            gather(2, 2)

            def inner(jj, carry2):
                j = jj * 4
                gather(j + 3, 3)
                drain(j, 0)
                gather(j + 4, 0)
                drain(j + 1, 1)
                gather(j + 5, 1)
                drain(j + 2, 2)
                gather(j + 6, 2)
                drain(j + 3, 3)
                return carry2

            lax.fori_loop(0, (IDX_BLK - 5) // 4, inner, 0)
            gather(IDX_BLK - 2, 3)
            drain(IDX_BLK - 5, 0)
            gather(IDX_BLK - 1, 0)
            drain(IDX_BLK - 4, 1)
            drain(IDX_BLK - 3, 2)
            drain(IDX_BLK - 2, 3)
            drain(IDX_BLK - 1, 0)
            return carry

        lax.fori_loop(0, N_OUTER, outer, 0)
        plsc.subcore_barrier()
        pltpu.sync_copy(accum.at[pl.ds(tbase, ROWS_T)],
                        agg_ref.at[pl.ds(tbase, ROWS_T)])

    @pl.when(c == 0)
    def _():
        run(y_lo, agg_lo)

    @pl.when(c == 1)
    def _():
        run(y_hi, agg_hi)


BLK = 2000
GRID = N_NODES // BLK

_row_spec = lambda w: pl.BlockSpec((BLK, w), lambda i: (i, 0))
_full_spec = lambda r, w: pl.BlockSpec((r, w), lambda i: (0, 0))


def _tc_proj_body(x_ref, win_ref, bin_ref, wc0_ref, deg_ref,
                  h_ref, dinv_ref, ylo_ref, yhi_ref):
    h = jnp.dot(x_ref[...], win_ref[...],
                preferred_element_type=jnp.float32) + bin_ref[...]
    dinv = lax.rsqrt(deg_ref[...][:, 0:1] + 1.0)
    y = jnp.dot(h, wc0_ref[...], preferred_element_type=jnp.float32) * dinv
    h_ref[...] = h
    dinv_ref[...] = dinv
    ylo_ref[...] = y[:, :HALF]
    yhi_ref[...] = y[:, HALF:]


_tc_proj = pl.pallas_call(
    _tc_proj_body,
    grid=(GRID,),
    in_specs=[_row_spec(5), _full_spec(5, HID), _full_spec(1, HID),
              _full_spec(HID, HID), _row_spec(DEGW)],
    out_specs=(_row_spec(HID), _row_spec(1), _row_spec(HALF), _row_spec(HALF)),
    out_shape=(jax.ShapeDtypeStruct((N_NODES, HID), jnp.float32),
               jax.ShapeDtypeStruct((N_NODES, 1), jnp.float32),
               jax.ShapeDtypeStruct((NPAD, HALF), jnp.float32),
               jax.ShapeDtypeStruct((NPAD, HALF), jnp.float32)),
)


def _layer_epilogue(h, agg, dinv, bc, g, be, relu):
    t = dinv * agg + bc
    mu = jnp.mean(t, axis=-1, keepdims=True)
    var = jnp.mean((t - mu) ** 2, axis=-1, keepdims=True)
    t = (t - mu) * lax.rsqrt(var + 1e-5) * g + be
    if relu:
        t = jnp.maximum(t, 0.0)
    return h + t


def _tc_mid_body(h_ref, alo_ref, ahi_ref, dinv_ref, bc_ref, g_ref, be_ref,
                 wn_ref, hn_ref, ylo_ref, yhi_ref):
    agg = jnp.concatenate([alo_ref[...], ahi_ref[...]], axis=1)
    dinv = dinv_ref[...]
    h_new = _layer_epilogue(h_ref[...], agg, dinv, bc_ref[...], g_ref[...],
                            be_ref[...], relu=True)
    y = jnp.dot(h_new, wn_ref[...], preferred_element_type=jnp.float32) * dinv
    hn_ref[...] = h_new
    ylo_ref[...] = y[:, :HALF]
    yhi_ref[...] = y[:, HALF:]


_tc_mid = pl.pallas_call(
    _tc_mid_body,
    grid=(GRID,),
    in_specs=[_row_spec(HID), _row_spec(HALF), _row_spec(HALF), _row_spec(1),
              _full_spec(1, HID), _full_spec(1, HID), _full_spec(1, HID),
              _full_spec(HID, HID)],
    out_specs=(_row_spec(HID), _row_spec(HALF), _row_spec(HALF)),
    out_shape=(jax.ShapeDtypeStruct((N_NODES, HID), jnp.float32),
               jax.ShapeDtypeStruct((NPAD, HALF), jnp.float32),
               jax.ShapeDtypeStruct((NPAD, HALF), jnp.float32)),
)


def _tc_fin_body(h_ref, alo_ref, ahi_ref, dinv_ref, bc_ref, g_ref, be_ref,
                 hn_ref):
    agg = jnp.concatenate([alo_ref[...], ahi_ref[...]], axis=1)
    hn_ref[...] = _layer_epilogue(h_ref[...], agg, dinv_ref[...], bc_ref[...],
                                  g_ref[...], be_ref[...], relu=False)


_tc_fin = pl.pallas_call(
    _tc_fin_body,
    grid=(GRID,),
    in_specs=[_row_spec(HID), _row_spec(HALF), _row_spec(HALF), _row_spec(1),
              _full_spec(1, HID), _full_spec(1, HID), _full_spec(1, HID)],
    out_specs=_row_spec(HID),
    out_shape=jax.ShapeDtypeStruct((N_NODES, HID), jnp.float32),
)


def kernel(x, edge_index, W_in, b_in, Wc0, bc0, g0, be0,
           Wc1, bc1, g1, be1, Wc2, bc2, g2, be2):
    epad = jnp.full((2, E_PAD - N_EDGES), NPAD - 1, jnp.int32)
    eip = jnp.concatenate([edge_index, epad], axis=1)
    src2 = eip[0].reshape(NUM_SUBCORES * N_OUTER, IDX_BLK, CHUNK)
    dst2 = eip[1].reshape(NUM_SUBCORES * N_OUTER, IDX_BLK, CHUNK)

    ones16 = jnp.ones((CHUNK, DEGW), jnp.float32)
    zeros16 = jnp.zeros((ROWS_T, DEGW), jnp.float32)
    deg = _sc_degree(dst2, ones16, zeros16)
    h, dinv, ylo, yhi = _tc_proj(x, W_in, b_in.reshape(1, HID), Wc0, deg)

    params = ((bc0, g0, be0, Wc1), (bc1, g1, be1, Wc2), (bc2, g2, be2, None))
    for i, (bc, g, be, wnext) in enumerate(params):
        agg_lo, agg_hi = _sc_segment(ylo, yhi, src2, dst2)
        bc2d, g2d, be2d = (bc.reshape(1, HID), g.reshape(1, HID),
                           be.reshape(1, HID))
        if wnext is not None:
            h, ylo, yhi = _tc_mid(h, agg_lo, agg_hi, dinv, bc2d, g2d, be2d,
                                  wnext)
        else:
            h = _tc_fin(h, agg_lo, agg_hi, dinv, bc2d, g2d, be2d)
    return h


# speedup vs baseline: 27.3786x; 1.0064x over previous
import functools

import jax
import jax.numpy as jnp
from jax import lax
from jax.experimental import pallas as pl
from jax.experimental.pallas import tpu as pltpu
from jax.experimental.pallas import tpu_sc as plsc

N_NODES = 50000
N_EDGES = 800000
HID = 64
HALF = 32

NUM_CORES = 2
NUM_SUBCORES = 16

CHUNK = 128
E_PAD = 802816
CHUNKS_TOTAL = E_PAD // CHUNK
CHUNKS_PER_TILE = CHUNKS_TOTAL // NUM_SUBCORES
IDX_BLK = 49
N_OUTER = CHUNKS_PER_TILE // IDX_BLK

NPAD = 50176
ROWS_T = NPAD // NUM_SUBCORES
ROWS_W = NPAD // (NUM_CORES * NUM_SUBCORES)

_MESH = plsc.VectorSubcoreMesh(
    core_axis_name="c", subcore_axis_name="s",
    num_cores=NUM_CORES, num_subcores=NUM_SUBCORES)

_SC_PARAMS = pltpu.CompilerParams(use_tc_tiling_on_sc=False)

DEGW = 16


@functools.partial(
    pl.kernel,
    out_type=(jax.ShapeDtypeStruct((NPAD, DEGW), jnp.float32),
              jax.ShapeDtypeStruct((NPAD, DEGW), jnp.float32)),
    mesh=_MESH,
    scratch_types=[
        pltpu.VMEM_SHARED((NPAD, DEGW), jnp.float32),
        pltpu.VMEM((IDX_BLK, CHUNK), jnp.int32),
        pltpu.VMEM((CHUNK, DEGW), jnp.float32),
    ],
    compiler_params=_SC_PARAMS,
)
def _sc_degree(dst2, ones_hbm, zeros_hbm, deg0_hbm, deg1_hbm,
               accum, didx, ones_v):
    c = lax.axis_index("c")
    s = lax.axis_index("s")

    pltpu.sync_copy(ones_hbm, ones_v)
    tbase = pl.multiple_of(s * ROWS_T, 8)
    pltpu.sync_copy(zeros_hbm, accum.at[pl.ds(tbase, ROWS_T)])
    plsc.subcore_barrier()

    def outer(ob, carry):
        blk = s * N_OUTER + c * (N_OUTER // 2) + ob
        pltpu.sync_copy(dst2.at[blk], didx)

        def inner(j, carry2):
            pltpu.sync_copy(ones_v, accum.at[didx.at[j]], add=True)
            return carry2

        lax.fori_loop(0, IDX_BLK, inner, 0)
        return carry

    lax.fori_loop(0, N_OUTER // 2, outer, 0)
    plsc.subcore_barrier()

    @pl.when(c == 0)
    def _():
        pltpu.sync_copy(accum.at[pl.ds(tbase, ROWS_T)],
                        deg0_hbm.at[pl.ds(tbase, ROWS_T)])

    @pl.when(c == 1)
    def _():
        pltpu.sync_copy(accum.at[pl.ds(tbase, ROWS_T)],
                        deg1_hbm.at[pl.ds(tbase, ROWS_T)])


@functools.partial(
    pl.kernel,
    out_type=(jax.ShapeDtypeStruct((NPAD, HALF), jnp.float32),
              jax.ShapeDtypeStruct((NPAD, HALF), jnp.float32)),
    mesh=_MESH,
    scratch_types=[
        pltpu.VMEM_SHARED((NPAD, HALF), jnp.float32),
        pltpu.VMEM((IDX_BLK, CHUNK), jnp.int32),
        pltpu.VMEM((IDX_BLK, CHUNK), jnp.int32),
        pltpu.VMEM((CHUNK, HALF), jnp.float32),
        pltpu.VMEM((CHUNK, HALF), jnp.float32),
        pltpu.VMEM((CHUNK, HALF), jnp.float32),
        pltpu.VMEM((CHUNK, HALF), jnp.float32),
        pltpu.SemaphoreType.DMA,
        pltpu.SemaphoreType.DMA,
        pltpu.SemaphoreType.DMA,
        pltpu.SemaphoreType.DMA,
    ],
    compiler_params=_SC_PARAMS,
)
def _sc_segment(y_lo, y_hi, src2, dst2, agg_lo, agg_hi, accum, sidx, didx,
                rows_a, rows_b, rows_c, rows_d, sem_a, sem_b, sem_c, sem_d):
    c = lax.axis_index("c")
    s = lax.axis_index("s")
    tbase = pl.multiple_of(s * ROWS_T, 8)
    bufs = ((rows_a, sem_a), (rows_b, sem_b), (rows_c, sem_c), (rows_d, sem_d))

    def run(y_ref, agg_ref):
        pltpu.sync_copy(y_ref.at[pl.ds(tbase, ROWS_T)],
                        accum.at[pl.ds(tbase, ROWS_T)])
        plsc.subcore_barrier()

        def gather(k, slot):
            buf, sem = bufs[slot]
            pltpu.async_copy(y_ref.at[sidx.at[k]], buf, sem)

        def drain(k, slot):
            buf, sem = bufs[slot]
            pltpu.make_async_copy(y_ref.at[sidx.at[k]], buf, sem).wait()
            pltpu.sync_copy(buf, accum.at[didx.at[k]], add=True)

        def outer(ob, carry):
            pltpu.sync_copy(src2.at[s * N_OUTER + ob], sidx)
            pltpu.sync_copy(dst2.at[s * N_OUTER + ob], didx)
            gather(0, 0)
            gather(1, 1)
            gather(2, 2)

            def inner(jj, carry2):
                j = jj * 4
                gather(j + 3, 3)
                drain(j, 0)
                gather(j + 4, 0)
                drain(j + 1, 1)
                gather(j + 5, 1)
                drain(j + 2, 2)
                gather(j + 6, 2)
                drain(j + 3, 3)
                return carry2

            lax.fori_loop(0, (IDX_BLK - 5) // 4, inner, 0)
            gather(IDX_BLK - 2, 3)
            drain(IDX_BLK - 5, 0)
            gather(IDX_BLK - 1, 0)
            drain(IDX_BLK - 4, 1)
            drain(IDX_BLK - 3, 2)
            drain(IDX_BLK - 2, 3)
            drain(IDX_BLK - 1, 0)
            return carry

        lax.fori_loop(0, N_OUTER, outer, 0)
        plsc.subcore_barrier()
        pltpu.sync_copy(accum.at[pl.ds(tbase, ROWS_T)],
                        agg_ref.at[pl.ds(tbase, ROWS_T)])

    @pl.when(c == 0)
    def _():
        run(y_lo, agg_lo)

    @pl.when(c == 1)
    def _():
        run(y_hi, agg_hi)


BLK = 2000
GRID = N_NODES // BLK

_row_spec = lambda w: pl.BlockSpec((BLK, w), lambda i: (i, 0))
_full_spec = lambda r, w: pl.BlockSpec((r, w), lambda i: (0, 0))


def _tc_proj_body(x_ref, win_ref, bin_ref, wc0_ref, deg0_ref, deg1_ref,
                  h_ref, dinv_ref, ylo_ref, yhi_ref):
    h = jnp.dot(x_ref[...], win_ref[...],
                preferred_element_type=jnp.float32) + bin_ref[...]
    dinv = lax.rsqrt(deg0_ref[...][:, 0:1] + deg1_ref[...][:, 0:1] + 1.0)
    y = jnp.dot(h, wc0_ref[...], preferred_element_type=jnp.float32) * dinv
    h_ref[...] = h
    dinv_ref[...] = dinv
    ylo_ref[...] = y[:, :HALF]
    yhi_ref[...] = y[:, HALF:]


_tc_proj = pl.pallas_call(
    _tc_proj_body,
    grid=(GRID,),
    in_specs=[_row_spec(5), _full_spec(5, HID), _full_spec(1, HID),
              _full_spec(HID, HID), _row_spec(DEGW), _row_spec(DEGW)],
    out_specs=(_row_spec(HID), _row_spec(1), _row_spec(HALF), _row_spec(HALF)),
    out_shape=(jax.ShapeDtypeStruct((N_NODES, HID), jnp.float32),
               jax.ShapeDtypeStruct((N_NODES, 1), jnp.float32),
               jax.ShapeDtypeStruct((NPAD, HALF), jnp.float32),
               jax.ShapeDtypeStruct((NPAD, HALF), jnp.float32)),
)


def _layer_epilogue(h, agg, dinv, bc, g, be, relu):
    t = dinv * agg + bc
    mu = jnp.mean(t, axis=-1, keepdims=True)
    var = jnp.mean((t - mu) ** 2, axis=-1, keepdims=True)
    t = (t - mu) * lax.rsqrt(var + 1e-5) * g + be
    if relu:
        t = jnp.maximum(t, 0.0)
    return h + t


def _tc_mid_body(h_ref, alo_ref, ahi_ref, dinv_ref, bc_ref, g_ref, be_ref,
                 wn_ref, hn_ref, ylo_ref, yhi_ref):
    agg = jnp.concatenate([alo_ref[...], ahi_ref[...]], axis=1)
    dinv = dinv_ref[...]
    h_new = _layer_epilogue(h_ref[...], agg, dinv, bc_ref[...], g_ref[...],
                            be_ref[...], relu=True)
    y = jnp.dot(h_new, wn_ref[...], preferred_element_type=jnp.float32) * dinv
    hn_ref[...] = h_new
    ylo_ref[...] = y[:, :HALF]
    yhi_ref[...] = y[:, HALF:]


_tc_mid = pl.pallas_call(
    _tc_mid_body,
    grid=(GRID,),
    in_specs=[_row_spec(HID), _row_spec(HALF), _row_spec(HALF), _row_spec(1),
              _full_spec(1, HID), _full_spec(1, HID), _full_spec(1, HID),
              _full_spec(HID, HID)],
    out_specs=(_row_spec(HID), _row_spec(HALF), _row_spec(HALF)),
    out_shape=(jax.ShapeDtypeStruct((N_NODES, HID), jnp.float32),
               jax.ShapeDtypeStruct((NPAD, HALF), jnp.float32),
               jax.ShapeDtypeStruct((NPAD, HALF), jnp.float32)),
)


def _tc_fin_body(h_ref, alo_ref, ahi_ref, dinv_ref, bc_ref, g_ref, be_ref,
                 hn_ref):
    agg = jnp.concatenate([alo_ref[...], ahi_ref[...]], axis=1)
    hn_ref[...] = _layer_epilogue(h_ref[...], agg, dinv_ref[...], bc_ref[...],
                                  g_ref[...], be_ref[...], relu=False)


_tc_fin = pl.pallas_call(
    _tc_fin_body,
    grid=(GRID,),
    in_specs=[_row_spec(HID), _row_spec(HALF), _row_spec(HALF), _row_spec(1),
              _full_spec(1, HID), _full_spec(1, HID), _full_spec(1, HID)],
    out_specs=_row_spec(HID),
    out_shape=jax.ShapeDtypeStruct((N_NODES, HID), jnp.float32),
)


def kernel(x, edge_index, W_in, b_in, Wc0, bc0, g0, be0,
           Wc1, bc1, g1, be1, Wc2, bc2, g2, be2):
    epad = jnp.full((2, E_PAD - N_EDGES), NPAD - 1, jnp.int32)
    eip = jnp.concatenate([edge_index, epad], axis=1)
    src2 = eip[0].reshape(NUM_SUBCORES * N_OUTER, IDX_BLK, CHUNK)
    dst2 = eip[1].reshape(NUM_SUBCORES * N_OUTER, IDX_BLK, CHUNK)

    ones16 = jnp.ones((CHUNK, DEGW), jnp.float32)
    zeros16 = jnp.zeros((ROWS_T, DEGW), jnp.float32)
    deg0, deg1 = _sc_degree(dst2, ones16, zeros16)
    h, dinv, ylo, yhi = _tc_proj(x, W_in, b_in.reshape(1, HID), Wc0,
                                 deg0, deg1)

    params = ((bc0, g0, be0, Wc1), (bc1, g1, be1, Wc2), (bc2, g2, be2, None))
    for i, (bc, g, be, wnext) in enumerate(params):
        agg_lo, agg_hi = _sc_segment(ylo, yhi, src2, dst2)
        bc2d, g2d, be2d = (bc.reshape(1, HID), g.reshape(1, HID),
                           be.reshape(1, HID))
        if wnext is not None:
            h, ylo, yhi = _tc_mid(h, agg_lo, agg_hi, dinv, bc2d, g2d, be2d,
                                  wnext)
        else:
            h = _tc_fin(h, agg_lo, agg_hi, dinv, bc2d, g2d, be2d)
    return h
